# baseline (speedup 1.0000x reference)
import functools

import jax
import jax.numpy as jnp
from jax import lax
from jax.experimental import pallas as pl
from jax.experimental.pallas import tpu as pltpu
from jax.experimental.pallas import tpu_sc as plsc

N = 10000
E = 320000
D = 128
G = 64
BN_EPS = 1e-5

NC = 2
NS = 16
NW = NC * NS
NBUF = 3
EPW = E // NW
CE = 80
CH = EPW // CE
GROUPS = (CH // NBUF)
ZROWS = 624
OROWS = 624
EPW_PAD = 10112
IDX_MASK = (1 << 14) - 1


def _agg_body(x_hbm, edges_hbm, zeros_hbm, out_hbm,
              idx_pk, idx_v, r0, r1, r2, agg_sh,
              g0, g1, g2, s0, s1, s2):
    rows = (r0, r1, r2)
    gsem = (g0, g1, g2)
    ssem = (s0, s1, s2)
    c = lax.axis_index("c")
    s = lax.axis_index("s")
    w = c * NS + s
    pltpu.sync_copy(edges_hbm.at[w], idx_pk)

    def unpack(j, b):
        for i in range(CE // 16):
            pk = idx_pk[pl.ds(j * CE + i * 16, 16)]
            idx_v[b, 0, pl.ds(i * 16, 16)] = pk & IDX_MASK
            idx_v[b, 1, pl.ds(i * 16, 16)] = lax.shift_right_logical(pk, 14)

    def gather_start(b):
        pltpu.async_copy(x_hbm.at[idx_v.at[b, 0]], rows[b], gsem[b])

    def gather_wait(b):
        pltpu.make_async_copy(x_hbm.at[idx_v.at[b, 0]], rows[b],
                              gsem[b]).wait()

    def scatter(b):
        pltpu.async_copy(rows[b], agg_sh.at[idx_v.at[b, 1]], ssem[b],
                         add=True).wait()

    for b in range(NBUF):
        unpack(jnp.int32(b), b)
        gather_start(b)
    pltpu.sync_copy(zeros_hbm, agg_sh.at[pl.ds(s * ZROWS, ZROWS)])

    @pl.when(s == 0)
    def _():
        pltpu.sync_copy(zeros_hbm.at[pl.ds(0, N - NS * ZROWS)],
                        agg_sh.at[pl.ds(NS * ZROWS, N - NS * ZROWS)])

    plsc.subcore_barrier()

    def group(g, carry):
        for k in range(NBUF):
            j = g * NBUF + k
            gather_wait(k)
            scatter(k)
            nj = j + NBUF

            @pl.when(nj < CH)
            def _():
                unpack(nj, k)
                gather_start(k)
        return carry

    lax.fori_loop(0, GROUPS, group, 0)
    for t in range(GROUPS * NBUF, CH):
        b = t % NBUF
        gather_wait(b)
        scatter(b)
    plsc.subcore_barrier()
    pltpu.sync_copy(agg_sh.at[pl.ds(s * OROWS, OROWS)],
                    out_hbm.at[c, pl.ds(s * OROWS, OROWS)])

    @pl.when(s == 0)
    def _():
        pltpu.sync_copy(agg_sh.at[pl.ds(NS * OROWS, N - NS * OROWS)],
                        out_hbm.at[c, pl.ds(NS * OROWS, N - NS * OROWS)])


_agg = pl.kernel(
    _agg_body,
    out_type=jax.ShapeDtypeStruct((NC, N, D), jnp.float32),
    mesh=plsc.VectorSubcoreMesh(core_axis_name="c", subcore_axis_name="s",
                                num_cores=NC, num_subcores=NS),
    scratch_types=[
        pltpu.VMEM((EPW,), jnp.int32),
        pltpu.VMEM((NBUF, 2, CE), jnp.int32),
        pltpu.VMEM((CE, D), jnp.float32),
        pltpu.VMEM((CE, D), jnp.float32),
        pltpu.VMEM((CE, D), jnp.float32),
        pltpu.VMEM_SHARED((N, D), jnp.float32),
        pltpu.SemaphoreType.DMA,
        pltpu.SemaphoreType.DMA,
        pltpu.SemaphoreType.DMA,
        pltpu.SemaphoreType.DMA,
        pltpu.SemaphoreType.DMA,
        pltpu.SemaphoreType.DMA,
    ],
)


def _mlp_body(x_ref, a_ref, ope_ref, w1_ref, b1_ref, w2_ref, b2_ref,
              g_ref, bt_ref, o_ref):
    h = x_ref[...] * ope_ref[...] + a_ref[0] + a_ref[1]
    h = jnp.maximum(
        jnp.dot(h, w1_ref[...], preferred_element_type=jnp.float32)
        + b1_ref[...], 0.0)
    h = jnp.dot(h, w2_ref[...], preferred_element_type=jnp.float32) + b2_ref[...]
    mu = jnp.mean(h, axis=0, keepdims=True)
    var = jnp.mean((h - mu) * (h - mu), axis=0, keepdims=True)
    h = (h - mu) * lax.rsqrt(var + BN_EPS) * g_ref[...] + bt_ref[...]
    o_ref[...] = jnp.maximum(h, 0.0)


_mlp = pl.pallas_call(
    _mlp_body,
    out_shape=jax.ShapeDtypeStruct((N, D), jnp.float32),
)


def _final_body(x_ref, a_ref, ope_ref, w1_ref, b1_ref, w2_ref, b2_ref,
                g_ref, bt_ref, batch_ref, hw1_ref, hb1_ref, hw2_ref, hb2_ref,
                o_ref):
    h = x_ref[...] * ope_ref[...] + a_ref[0] + a_ref[1]
    h = jnp.maximum(
        jnp.dot(h, w1_ref[...], preferred_element_type=jnp.float32)
        + b1_ref[...], 0.0)
    h = jnp.dot(h, w2_ref[...], preferred_element_type=jnp.float32) + b2_ref[...]
    mu = jnp.mean(h, axis=0, keepdims=True)
    var = jnp.mean((h - mu) * (h - mu), axis=0, keepdims=True)
    h = (h - mu) * lax.rsqrt(var + BN_EPS) * g_ref[...] + bt_ref[...]
    h = jnp.maximum(h, 0.0)
    gi = lax.broadcasted_iota(jnp.int32, (N, G), 1)
    oh = (batch_ref[...] == gi).astype(jnp.float32)
    s = lax.dot_general(oh, h, (((0,), (0,)), ((), ())),
                        preferred_element_type=jnp.float32)
    cnt = lax.dot_general(oh, jnp.ones((N, 1), jnp.float32),
                          (((0,), (0,)), ((), ())),
                          preferred_element_type=jnp.float32)
    pooled = s / jnp.maximum(cnt, 1.0)
    hh = jnp.maximum(
        jnp.dot(pooled, hw1_ref[...], preferred_element_type=jnp.float32)
        + hb1_ref[...], 0.0)
    o_ref[...] = (jnp.dot(hh, hw2_ref[...], preferred_element_type=jnp.float32)
                  + hb2_ref[...])


_final = pl.pallas_call(
    _final_body,
    out_shape=jax.ShapeDtypeStruct((G, D), jnp.float32),
)


def kernel(x, edge_index, batch,
           eps_0, w1_0, b1_0, w2_0, b2_0, g_0, bt_0,
           eps_1, w1_1, b1_1, w2_1, b2_1, g_1, bt_1,
           eps_2, w1_2, b1_2, w2_2, b2_2, g_2, bt_2,
           hw1, hb1, hw2, hb2):
    packed = edge_index[0] | (edge_index[1] << 14)
    edges_p = packed.reshape(NW, EPW)
    zeros = jnp.zeros((ZROWS, D), jnp.float32)
    batch2d = batch.reshape(N, 1)

    layers = [
        (eps_0, w1_0, b1_0, w2_0, b2_0, g_0, bt_0),
        (eps_1, w1_1, b1_1, w2_1, b2_1, g_1, bt_1),
        (eps_2, w1_2, b1_2, w2_2, b2_2, g_2, bt_2),
    ]

    for l, (eps, w1, b1, w2, b2, g, bt) in enumerate(layers):
        a = _agg(x, edges_p, zeros)
        ope = (1.0 + eps).reshape(1, 1).astype(jnp.float32)
        args = (x, a, ope, w1, b1.reshape(1, D), w2, b2.reshape(1, D),
                g.reshape(1, D), bt.reshape(1, D))
        if l < 2:
            x = _mlp(*args)
        else:
            return _final(*args, batch2d, hw1, hb1.reshape(1, D),
                          hw2, hb2.reshape(1, D))


# speedup vs baseline: 1.0078x; 1.0078x over previous
import functools

import jax
import jax.numpy as jnp
from jax import lax
from jax.experimental import pallas as pl
from jax.experimental.pallas import tpu as pltpu
from jax.experimental.pallas import tpu_sc as plsc

N = 10000
E = 320000
D = 128
G = 64
BN_EPS = 1e-5

NC = 2
NS = 16
NW = NC * NS
NBUF = 3
EPW = E // NW
CE = 80
CH = EPW // CE
GROUPS = (CH // NBUF)
ZROWS = 624
OROWS = 624
EPW_PAD = 10112
IDX_MASK = (1 << 14) - 1


def _agg_body(x_hbm, edges_hbm, zeros_hbm, out_hbm,
              idx_pk, idx_v, r0, r1, r2, agg_sh,
              g0, g1, g2, s0, s1, s2):
    rows = (r0, r1, r2)
    gsem = (g0, g1, g2)
    ssem = (s0, s1, s2)
    c = lax.axis_index("c")
    s = lax.axis_index("s")
    w = c * NS + s
    pltpu.sync_copy(edges_hbm.at[w], idx_pk)

    def unpack(j, b):
        for i in range(CE // 16):
            pk = idx_pk[pl.ds(j * CE + i * 16, 16)]
            idx_v[b, 0, pl.ds(i * 16, 16)] = pk & IDX_MASK
            idx_v[b, 1, pl.ds(i * 16, 16)] = lax.shift_right_logical(pk, 14)

    def gather_start(b):
        pltpu.async_copy(x_hbm.at[idx_v.at[b, 0]], rows[b], gsem[b])

    def gather_wait(b):
        pltpu.make_async_copy(x_hbm.at[idx_v.at[b, 0]], rows[b],
                              gsem[b]).wait()

    def scatter(b):
        pltpu.async_copy(rows[b], agg_sh.at[idx_v.at[b, 1]], ssem[b],
                         add=True).wait()

    for b in range(NBUF):
        unpack(jnp.int32(b), b)
        gather_start(b)
    pltpu.sync_copy(zeros_hbm, agg_sh.at[pl.ds(s * ZROWS, ZROWS)])

    @pl.when(s == 0)
    def _():
        pltpu.sync_copy(zeros_hbm.at[pl.ds(0, N - NS * ZROWS)],
                        agg_sh.at[pl.ds(NS * ZROWS, N - NS * ZROWS)])

    plsc.subcore_barrier()

    def group(g, carry):
        for k in range(NBUF):
            j = g * NBUF + k
            gather_wait(k)
            scatter(k)
            nj = j + NBUF

            @pl.when(nj < CH)
            def _():
                unpack(nj, k)
                gather_start(k)
        return carry

    lax.fori_loop(0, GROUPS, group, 0)
    for t in range(GROUPS * NBUF, CH):
        b = t % NBUF
        gather_wait(b)
        scatter(b)
    plsc.subcore_barrier()
    pltpu.sync_copy(agg_sh.at[pl.ds(s * OROWS, OROWS)],
                    out_hbm.at[c, pl.ds(s * OROWS, OROWS)])

    @pl.when(s == 0)
    def _():
        pltpu.sync_copy(agg_sh.at[pl.ds(NS * OROWS, N - NS * OROWS)],
                        out_hbm.at[c, pl.ds(NS * OROWS, N - NS * OROWS)])


_agg = pl.kernel(
    _agg_body,
    out_type=jax.ShapeDtypeStruct((NC, N, D), jnp.float32),
    mesh=plsc.VectorSubcoreMesh(core_axis_name="c", subcore_axis_name="s",
                                num_cores=NC, num_subcores=NS),
    scratch_types=[
        pltpu.VMEM((EPW,), jnp.int32),
        pltpu.VMEM((NBUF, 2, CE), jnp.int32),
        pltpu.VMEM((CE, D), jnp.float32),
        pltpu.VMEM((CE, D), jnp.float32),
        pltpu.VMEM((CE, D), jnp.float32),
        pltpu.VMEM_SHARED((N, D), jnp.float32),
        pltpu.SemaphoreType.DMA,
        pltpu.SemaphoreType.DMA,
        pltpu.SemaphoreType.DMA,
        pltpu.SemaphoreType.DMA,
        pltpu.SemaphoreType.DMA,
        pltpu.SemaphoreType.DMA,
    ],
)


def _mlp_body(x_ref, a_ref, ope_ref, w1_ref, b1_ref, w2_ref, b2_ref,
              g_ref, bt_ref, o_ref):
    h = x_ref[...] * ope_ref[...] + a_ref[0] + a_ref[1]
    h = jnp.maximum(
        jnp.dot(h, w1_ref[...], preferred_element_type=jnp.float32)
        + b1_ref[...], 0.0)
    h = jnp.dot(h, w2_ref[...], preferred_element_type=jnp.float32) + b2_ref[...]
    mu = jnp.mean(h, axis=0, keepdims=True)
    var = jnp.mean((h - mu) * (h - mu), axis=0, keepdims=True)
    h = (h - mu) * lax.rsqrt(var + BN_EPS) * g_ref[...] + bt_ref[...]
    o_ref[...] = jnp.maximum(h, 0.0)


_mlp = pl.pallas_call(
    _mlp_body,
    out_shape=jax.ShapeDtypeStruct((N, D), jnp.float32),
)


def _final_body(x_ref, a_ref, ope_ref, w1_ref, b1_ref, w2_ref, b2_ref,
                g_ref, bt_ref, batch_ref, hw1_ref, hb1_ref, hw2_ref, hb2_ref,
                o_ref):
    h = x_ref[...] * ope_ref[...] + a_ref[0] + a_ref[1]
    h = jnp.maximum(
        jnp.dot(h, w1_ref[...], preferred_element_type=jnp.float32)
        + b1_ref[...], 0.0)
    h = jnp.dot(h, w2_ref[...], preferred_element_type=jnp.float32) + b2_ref[...]
    mu = jnp.mean(h, axis=0, keepdims=True)
    var = jnp.mean((h - mu) * (h - mu), axis=0, keepdims=True)
    h = (h - mu) * lax.rsqrt(var + BN_EPS) * g_ref[...] + bt_ref[...]
    h = jnp.maximum(h, 0.0)
    gi = lax.broadcasted_iota(jnp.int32, (G, N), 0)
    oh = (batch_ref[...] == gi).astype(jnp.float32)
    s = jnp.dot(oh, h, preferred_element_type=jnp.float32)
    cnt = jnp.dot(oh, jnp.ones((N, 1), jnp.float32),
                  preferred_element_type=jnp.float32)
    pooled = s / jnp.maximum(cnt, 1.0)
    hh = jnp.maximum(
        jnp.dot(pooled, hw1_ref[...], preferred_element_type=jnp.float32)
        + hb1_ref[...], 0.0)
    o_ref[...] = (jnp.dot(hh, hw2_ref[...], preferred_element_type=jnp.float32)
                  + hb2_ref[...])


_final = pl.pallas_call(
    _final_body,
    out_shape=jax.ShapeDtypeStruct((G, D), jnp.float32),
)


def kernel(x, edge_index, batch,
           eps_0, w1_0, b1_0, w2_0, b2_0, g_0, bt_0,
           eps_1, w1_1, b1_1, w2_1, b2_1, g_1, bt_1,
           eps_2, w1_2, b1_2, w2_2, b2_2, g_2, bt_2,
           hw1, hb1, hw2, hb2):
    packed = edge_index[0] | (edge_index[1] << 14)
    edges_p = packed.reshape(NW, EPW)
    zeros = jnp.zeros((ZROWS, D), jnp.float32)
    batch2d = batch.reshape(1, N)

    layers = [
        (eps_0, w1_0, b1_0, w2_0, b2_0, g_0, bt_0),
        (eps_1, w1_1, b1_1, w2_1, b2_1, g_1, bt_1),
        (eps_2, w1_2, b1_2, w2_2, b2_2, g_2, bt_2),
    ]

    for l, (eps, w1, b1, w2, b2, g, bt) in enumerate(layers):
        a = _agg(x, edges_p, zeros)
        ope = (1.0 + eps).reshape(1, 1).astype(jnp.float32)
        args = (x, a, ope, w1, b1.reshape(1, D), w2, b2.reshape(1, D),
                g.reshape(1, D), bt.reshape(1, D))
        if l < 2:
            x = _mlp(*args)
        else:
            return _final(*args, batch2d, hw1, hb1.reshape(1, D),
                          hw2, hb2.reshape(1, D))
